# Initial kernel scaffold; baseline (speedup 1.0000x reference)
import jax
import jax.numpy as jnp
from jax.experimental import pallas as pl


def kernel(x, t, centers, u, R):
    raise NotImplementedError("write your pallas kernel here")


# speedup vs baseline: 1.0223x; 1.0223x over previous
import functools

import jax
import jax.numpy as jnp
from jax.experimental import pallas as pl
from jax.experimental.pallas import tpu as pltpu

_NL = 30
_NS = 15
_N = 4096
_BK = 256
_BA = 256


def _knn_body(xt_ref, c_ref, idx_ref, d2_ref):
    b = xt_ref.shape[1]
    n = c_ref.shape[0]
    x0 = xt_ref[0:1, :]
    x1 = xt_ref[1:2, :]
    c0 = c_ref[:, 0:1]
    c1 = c_ref[:, 1:2]
    dx = c0 - x0
    dy = c1 - x1
    d2_ref[...] = dx * dx + dy * dy
    iota = jax.lax.broadcasted_iota(jnp.int32, (n, b), 0)
    kiota = jax.lax.broadcasted_iota(jnp.int32, (32, b), 0)

    def body(k, acc):
        d2 = d2_ref[...]
        m = jnp.min(d2, axis=0, keepdims=True)
        j = jnp.min(jnp.where(d2 == m, iota, n), axis=0, keepdims=True)
        acc = jnp.where(kiota == k, j, acc)
        d2_ref[...] = jnp.where(iota == j, jnp.float32(jnp.inf), d2)
        return acc

    acc = jnp.zeros((32, b), jnp.int32)
    idx_ref[...] = jax.lax.fori_loop(0, _NL + 1, body, acc)


def _knn(xt, centers):
    grid = _N // _BK
    idx = pl.pallas_call(
        _knn_body,
        grid=(grid,),
        in_specs=[
            pl.BlockSpec((2, _BK), lambda i: (0, i)),
            pl.BlockSpec((_N, 2), lambda i: (0, 0)),
        ],
        out_specs=pl.BlockSpec((32, _BK), lambda i: (0, i)),
        out_shape=jax.ShapeDtypeStruct((32, _N), jnp.int32),
        scratch_shapes=[pltpu.VMEM((_N, _BK), jnp.float32)],
    )(xt, centers)
    return idx


def _assemble_body(vals_ref, idx_ref, fa_ref):
    b, n = fa_ref.shape
    gi = pl.program_id(0)
    col = jax.lax.broadcasted_iota(jnp.int32, (b, n), 1)
    row = jax.lax.broadcasted_iota(jnp.int32, (b, n), 0) + gi * b
    acc = jnp.where(col == row, jnp.float32(1.0), jnp.float32(0.0))
    for l in range(_NL):
        cl = idx_ref[:, l : l + 1]
        vl = vals_ref[:, l : l + 1]
        acc = acc + jnp.where(col == cl, vl, jnp.float32(0.0))
    fa_ref[...] = acc


def _assemble(vals, idx):
    grid = _N // _BA
    return pl.pallas_call(
        _assemble_body,
        grid=(grid,),
        in_specs=[
            pl.BlockSpec((_BA, 32), lambda i: (i, 0)),
            pl.BlockSpec((_BA, 32), lambda i: (i, 0)),
        ],
        out_specs=pl.BlockSpec((_BA, _N), lambda i: (i, 0)),
        out_shape=jax.ShapeDtypeStruct((_N, _N), jnp.float32),
    )(vals, idx)


@jax.jit
def _run(x, centers, u, R):
    n = _N
    idx_raw = _knn(x.T, centers).T
    idx = jnp.sort(idx_raw[:, 1 : _NL + 1], axis=1)

    tpx = centers[idx]
    sinta = jnp.broadcast_to(
        (jnp.arange(_NS, dtype=x.dtype) / _NS)[None, :], (n, _NS)
    ) * 2.0 * jnp.pi
    maxdm = jnp.max(jnp.sqrt(jnp.sum((x[:, None, :] - tpx) ** 2, axis=-1)))
    rxy = jnp.concatenate(
        [jnp.cos(sinta)[:, :, None], jnp.sin(sinta)[:, :, None]], axis=-1
    )
    c_r = R[:, :, None] * maxdm * rxy
    c = centers[:, None, :] + c_r
    dm = jnp.sqrt(jnp.sum((x[:, None, :] - c) ** 2, axis=-1))[:, None, :]
    dm1 = jnp.transpose(
        jnp.sqrt(jnp.sum((tpx[:, None, :, :] - c[:, :, None, :]) ** 2, axis=-1)),
        (0, 2, 1),
    )
    mfs = jnp.log(dm)
    mfs1 = jnp.log(dm1)
    mfs_pinv = jnp.linalg.pinv(mfs1)
    vals = -jnp.matmul(mfs, mfs_pinv)[:, 0, :]

    pad_i = jnp.full((n, 32 - _NL), n + 7, dtype=jnp.int32)
    pad_v = jnp.zeros((n, 32 - _NL), dtype=jnp.float32)
    idx32 = jnp.concatenate([idx.astype(jnp.int32), pad_i], axis=1)
    vals32 = jnp.concatenate([vals, pad_v], axis=1)

    f_A = _assemble(vals32, idx32)

    un = u[idx, 0]
    f = u[:, 0] + jnp.sum(vals * un, axis=1)
    return f, u[:, 0], f_A


def kernel(x, t, centers, u, R):
    return _run(x, centers, u, R)


# speedup vs baseline: 39.6054x; 38.7404x over previous
import functools

import jax
import jax.numpy as jnp
from jax.experimental import pallas as pl
from jax.experimental.pallas import tpu as pltpu

_NL = 30
_NS = 15
_N = 4096
_BK = 256
_BA = 256
_BJ = 1024
_SWEEPS = 6
_RTOL = 10.0 * _NL * float(jnp.finfo(jnp.float32).eps)


def _knn_body(xt_ref, c_ref, idx_ref, d2_ref):
    b = xt_ref.shape[1]
    n = c_ref.shape[0]
    x0 = xt_ref[0:1, :]
    x1 = xt_ref[1:2, :]
    c0 = c_ref[:, 0:1]
    c1 = c_ref[:, 1:2]
    dx = c0 - x0
    dy = c1 - x1
    d2_ref[...] = dx * dx + dy * dy
    iota = jax.lax.broadcasted_iota(jnp.int32, (n, b), 0)
    kiota = jax.lax.broadcasted_iota(jnp.int32, (32, b), 0)

    def body(k, acc):
        d2 = d2_ref[...]
        m = jnp.min(d2, axis=0, keepdims=True)
        j = jnp.min(jnp.where(d2 == m, iota, n), axis=0, keepdims=True)
        acc = jnp.where(kiota == k, j, acc)
        d2_ref[...] = jnp.where(iota == j, jnp.float32(jnp.inf), d2)
        return acc

    acc = jnp.zeros((32, b), jnp.int32)
    idx_ref[...] = jax.lax.fori_loop(0, _NL + 1, body, acc)


def _knn(xt, centers):
    grid = _N // _BK
    idx = pl.pallas_call(
        _knn_body,
        grid=(grid,),
        in_specs=[
            pl.BlockSpec((2, _BK), lambda i: (0, i)),
            pl.BlockSpec((_N, 2), lambda i: (0, 0)),
        ],
        out_specs=pl.BlockSpec((32, _BK), lambda i: (0, i)),
        out_shape=jax.ShapeDtypeStruct((32, _N), jnp.int32),
        scratch_shapes=[pltpu.VMEM((_N, _BK), jnp.float32)],
    )(xt, centers)
    return idx


def _jacobi_body(a_ref, b_ref, vals_ref, w_ref, v_ref):
    b = a_ref.shape[2]
    one = jnp.float32(1.0)
    zero = jnp.float32(0.0)
    w_ref[...] = a_ref[...]
    vinit = jnp.zeros((_NS, _NS, b), jnp.float32)
    eye = jax.lax.broadcasted_iota(jnp.int32, (_NS, _NS, b), 0) == (
        jax.lax.broadcasted_iota(jnp.int32, (_NS, _NS, b), 1)
    )
    v_ref[...] = jnp.where(eye, one, vinit)

    def sweep(_, carry):
        for p in range(_NS - 1):
            for q in range(p + 1, _NS):
                wp = w_ref[p]
                wq = w_ref[q]
                alpha = jnp.sum(wp * wp, axis=0, keepdims=True)
                beta = jnp.sum(wq * wq, axis=0, keepdims=True)
                gamma = jnp.sum(wp * wq, axis=0, keepdims=True)
                zeta = (beta - alpha) / (2.0 * gamma)
                sgn = jnp.where(zeta >= 0, one, -one)
                t = -sgn / (jnp.abs(zeta) + jnp.sqrt(one + zeta * zeta))
                t = jnp.where(gamma == 0, zero, t)
                cth = one / jnp.sqrt(one + t * t)
                sth = cth * t
                w_ref[p] = cth * wp + sth * wq
                w_ref[q] = cth * wq - sth * wp
                vp = v_ref[p]
                vq = v_ref[q]
                v_ref[p] = cth * vp + sth * vq
                v_ref[q] = cth * vq - sth * vp
        return carry

    jax.lax.fori_loop(0, _SWEEPS, sweep, 0)

    s2 = []
    for k in range(_NS):
        wk = w_ref[k]
        s2.append(jnp.sum(wk * wk, axis=0, keepdims=True))
    s2max = s2[0]
    for k in range(1, _NS):
        s2max = jnp.maximum(s2max, s2[k])
    cut2 = jnp.float32(_RTOL * _RTOL) * s2max
    bvec = b_ref[...]
    acc = jnp.zeros((_NL, b), jnp.float32)
    for k in range(_NS):
        rk = jnp.sum(v_ref[k] * bvec, axis=0, keepdims=True)
        coef = jnp.where(s2[k] > cut2, rk / s2[k], zero)
        acc = acc - w_ref[k] * coef
    vals_ref[...] = jnp.concatenate([acc, jnp.zeros((2, b), jnp.float32)], axis=0)


def _jacobi_vals(a_t, b_t):
    grid = _N // _BJ
    return pl.pallas_call(
        _jacobi_body,
        grid=(grid,),
        in_specs=[
            pl.BlockSpec((_NS, _NL, _BJ), lambda i: (0, 0, i)),
            pl.BlockSpec((_NS, _BJ), lambda i: (0, i)),
        ],
        out_specs=pl.BlockSpec((32, _BJ), lambda i: (0, i)),
        out_shape=jax.ShapeDtypeStruct((32, _N), jnp.float32),
        scratch_shapes=[
            pltpu.VMEM((_NS, _NL, _BJ), jnp.float32),
            pltpu.VMEM((_NS, _NS, _BJ), jnp.float32),
        ],
    )(a_t, b_t)


def _assemble_body(vals_ref, idx_ref, fa_ref):
    b, n = fa_ref.shape
    gi = pl.program_id(0)
    col = jax.lax.broadcasted_iota(jnp.int32, (b, n), 1)
    row = jax.lax.broadcasted_iota(jnp.int32, (b, n), 0) + gi * b
    acc = jnp.where(col == row, jnp.float32(1.0), jnp.float32(0.0))
    for l in range(_NL):
        cl = idx_ref[:, l : l + 1]
        vl = vals_ref[:, l : l + 1]
        acc = acc + jnp.where(col == cl, vl, jnp.float32(0.0))
    fa_ref[...] = acc


def _assemble(vals, idx):
    grid = _N // _BA
    return pl.pallas_call(
        _assemble_body,
        grid=(grid,),
        in_specs=[
            pl.BlockSpec((_BA, 32), lambda i: (i, 0)),
            pl.BlockSpec((_BA, 32), lambda i: (i, 0)),
        ],
        out_specs=pl.BlockSpec((_BA, _N), lambda i: (i, 0)),
        out_shape=jax.ShapeDtypeStruct((_N, _N), jnp.float32),
    )(vals, idx)


@jax.jit
def _run(x, centers, u, R):
    n = _N
    idx_raw = _knn(x.T, centers).T
    idx = jnp.sort(idx_raw[:, 1 : _NL + 1], axis=1)

    tpx = centers[idx]
    sinta = jnp.broadcast_to(
        (jnp.arange(_NS, dtype=x.dtype) / _NS)[None, :], (n, _NS)
    ) * 2.0 * jnp.pi
    maxdm = jnp.max(jnp.sqrt(jnp.sum((x[:, None, :] - tpx) ** 2, axis=-1)))
    rxy = jnp.concatenate(
        [jnp.cos(sinta)[:, :, None], jnp.sin(sinta)[:, :, None]], axis=-1
    )
    c_r = R[:, :, None] * maxdm * rxy
    c = centers[:, None, :] + c_r
    dm = jnp.sqrt(jnp.sum((x[:, None, :] - c) ** 2, axis=-1))[:, None, :]
    dm1 = jnp.transpose(
        jnp.sqrt(jnp.sum((tpx[:, None, :, :] - c[:, :, None, :]) ** 2, axis=-1)),
        (0, 2, 1),
    )
    mfs = jnp.log(dm)
    mfs1 = jnp.log(dm1)
    a_t = jnp.transpose(mfs1, (2, 1, 0))
    b_t = mfs[:, 0, :].T
    vals = _jacobi_vals(a_t, b_t).T[:, :_NL]

    pad_i = jnp.full((n, 32 - _NL), n + 7, dtype=jnp.int32)
    pad_v = jnp.zeros((n, 32 - _NL), dtype=jnp.float32)
    idx32 = jnp.concatenate([idx.astype(jnp.int32), pad_i], axis=1)
    vals32 = jnp.concatenate([vals, pad_v], axis=1)

    f_A = _assemble(vals32, idx32)

    un = u[idx, 0]
    f = u[:, 0] + jnp.sum(vals * un, axis=1)
    return f, u[:, 0], f_A


def kernel(x, t, centers, u, R):
    return _run(x, centers, u, R)
